# Initial kernel scaffold; baseline (speedup 1.0000x reference)
import jax
import jax.numpy as jnp
from jax.experimental import pallas as pl


def kernel(frame_indices, time_emb_weight):
    raise NotImplementedError("write your pallas kernel here")


# speedup vs baseline: 1.2511x; 1.2511x over previous
import functools

import jax
import jax.numpy as jnp
from jax import lax
from jax.experimental import pallas as pl
from jax.experimental.pallas import tpu as pltpu
from jax.experimental.pallas import tpu_sc as plsc

MAX_FRAMES = 256
DIM = 1536
LANES = 16
CHUNK = 32


@functools.cache
def _num_workers():
    try:
        info = plsc.get_sparse_core_info()
        return int(info.num_cores), int(info.num_subcores)
    except Exception:
        return 2, 16


@functools.cache
def _build(total_rows):
    nc, ns = _num_workers()
    nw = nc * ns
    per_worker = total_rows // nw
    n_chunks = per_worker // CHUNK
    mesh = plsc.VectorSubcoreMesh(core_axis_name="c", subcore_axis_name="s")

    @functools.partial(
        pl.kernel,
        mesh=mesh,
        out_type=jax.ShapeDtypeStruct((total_rows, DIM), jnp.float32),
        scratch_types=[
            pltpu.VMEM((n_chunks, CHUNK), jnp.int32),
            pltpu.VMEM((CHUNK, DIM), jnp.float32),
            pltpu.SemaphoreType.DMA,
        ],
    )
    def emb_kernel(table_hbm, idx_hbm, out_hbm, idx_v, rows_v, sem):
        wid = lax.axis_index("s") * nc + lax.axis_index("c")
        pltpu.sync_copy(idx_hbm.at[wid], idx_v)

        def clamp_row(j, carry):
            for k in range(CHUNK // LANES):
                v = idx_v[j, pl.ds(k * LANES, LANES)]
                v = jnp.minimum(jnp.maximum(v, 0), MAX_FRAMES - 1)
                idx_v[j, pl.ds(k * LANES, LANES)] = v
            return carry

        lax.fori_loop(0, n_chunks, clamp_row, 0)

        base = wid * per_worker

        def chunk(j, carry):
            pltpu.async_copy(table_hbm.at[idx_v.at[j]], rows_v, sem).wait()
            pltpu.sync_copy(rows_v, out_hbm.at[pl.ds(base + j * CHUNK, CHUNK)])
            return carry

        lax.fori_loop(0, n_chunks, chunk, 0)

    return emb_kernel


def kernel(frame_indices, time_emb_weight):
    b, t = frame_indices.shape
    total = b * t
    nc, ns = _num_workers()
    nw = nc * ns
    idx = frame_indices.reshape(-1).astype(jnp.int32)
    idx = idx.reshape(nw, total // nw // CHUNK, CHUNK)
    out = _build(total)(time_emb_weight, idx)
    return out.reshape(b, t, DIM)


# speedup vs baseline: 1.2644x; 1.0106x over previous
import functools

import jax
import jax.numpy as jnp
from jax import lax
from jax.experimental import pallas as pl
from jax.experimental.pallas import tpu as pltpu
from jax.experimental.pallas import tpu_sc as plsc

MAX_FRAMES = 256
DIM = 1536
LANES = 16
CHUNK = 16
NBUF = 4


@functools.cache
def _num_workers():
    try:
        info = plsc.get_sparse_core_info()
        return int(info.num_cores), int(info.num_subcores)
    except Exception:
        return 2, 16


@functools.cache
def _build(total_rows):
    nc, ns = _num_workers()
    nw = nc * ns
    per_worker = total_rows // nw
    n_chunks = per_worker // CHUNK
    mesh = plsc.VectorSubcoreMesh(core_axis_name="c", subcore_axis_name="s")

    @functools.partial(
        pl.kernel,
        mesh=mesh,
        out_type=jax.ShapeDtypeStruct((total_rows, DIM), jnp.float32),
        scratch_types=[
            pltpu.VMEM((n_chunks, CHUNK), jnp.int32),
            [pltpu.VMEM((CHUNK, DIM), jnp.float32) for _ in range(NBUF)],
            [pltpu.SemaphoreType.DMA for _ in range(NBUF)],
            [pltpu.SemaphoreType.DMA for _ in range(NBUF)],
        ],
    )
    def emb_kernel(table_hbm, idx_hbm, out_hbm, idx_v, rows, gsem, osem):
        wid = lax.axis_index("s") * nc + lax.axis_index("c")
        pltpu.sync_copy(idx_hbm.at[wid], idx_v)

        def clamp_row(j, carry):
            v = idx_v[j]
            idx_v[j] = jnp.minimum(jnp.maximum(v, 0), MAX_FRAMES - 1)
            return carry

        lax.fori_loop(0, n_chunks, clamp_row, 0)

        base = wid * per_worker

        def start_gather(j, b):
            pltpu.async_copy(table_hbm.at[idx_v.at[j]], rows[b], gsem[b])

        def wait_gather(j, b):
            pltpu.make_async_copy(table_hbm.at[idx_v.at[j]], rows[b], gsem[b]).wait()

        def out_slice(j):
            return out_hbm.at[pl.ds(base + j * CHUNK, CHUNK)]

        def start_out(j, b):
            pltpu.async_copy(rows[b], out_slice(j), osem[b])

        def wait_out(j, b):
            pltpu.make_async_copy(rows[b], out_slice(j), osem[b]).wait()

        for b in range(NBUF):
            start_gather(b, b)

        def body(jj, carry):
            j0 = jj * NBUF
            for b in range(NBUF):
                wait_gather(j0 + b, b)
                start_out(j0 + b, b)
            for b in range(NBUF):
                wait_out(j0 + b, b)
                start_gather(j0 + NBUF + b, b)
            return carry

        lax.fori_loop(0, n_chunks // NBUF - 1, body, 0)

        jlast = n_chunks - NBUF
        for b in range(NBUF):
            wait_gather(jlast + b, b)
            start_out(jlast + b, b)
        for b in range(NBUF):
            wait_out(jlast + b, b)

    return emb_kernel


def kernel(frame_indices, time_emb_weight):
    b, t = frame_indices.shape
    total = b * t
    nc, ns = _num_workers()
    nw = nc * ns
    idx = frame_indices.reshape(-1).astype(jnp.int32)
    idx = idx.reshape(nw, total // nw // CHUNK, CHUNK)
    out = _build(total)(time_emb_weight, idx)
    return out.reshape(b, t, DIM)


# speedup vs baseline: 3.6175x; 2.8610x over previous
import functools

import jax
import jax.numpy as jnp
from jax import lax
from jax.experimental import pallas as pl
from jax.experimental.pallas import tpu as pltpu
from jax.experimental.pallas import tpu_sc as plsc

MAX_FRAMES = 256
DIM = 1536
LANES = 16
NBUF = 2


@functools.cache
def _num_workers():
    try:
        info = plsc.get_sparse_core_info()
        return int(info.num_cores), int(info.num_subcores)
    except Exception:
        return 2, 16


@functools.cache
def _build(batch, seq):
    nc, ns = _num_workers()
    nw = nc * ns
    bpw = batch // nw
    mesh = plsc.VectorSubcoreMesh(core_axis_name="c", subcore_axis_name="s")

    @functools.partial(
        pl.kernel,
        mesh=mesh,
        out_type=jax.ShapeDtypeStruct((seq, batch, DIM), jnp.float32),
        scratch_types=[
            pltpu.VMEM((seq, bpw), jnp.int32),
            [pltpu.VMEM((bpw, DIM), jnp.float32) for _ in range(NBUF)],
            [pltpu.SemaphoreType.DMA for _ in range(NBUF)],
            [pltpu.SemaphoreType.DMA for _ in range(NBUF)],
        ],
    )
    def emb_kernel(table_hbm, idx_hbm, out_hbm, idx_v, rows, gsem, osem):
        wid = lax.axis_index("s") * nc + lax.axis_index("c")
        pltpu.sync_copy(idx_hbm.at[wid], idx_v)

        def clamp_row(j, carry):
            for k in range(bpw // LANES):
                v = idx_v[j, pl.ds(k * LANES, LANES)]
                idx_v[j, pl.ds(k * LANES, LANES)] = jnp.minimum(
                    jnp.maximum(v, 0), MAX_FRAMES - 1
                )
            return carry

        lax.fori_loop(0, seq, clamp_row, 0)

        base = wid * bpw

        def start_gather(j, b):
            pltpu.async_copy(table_hbm.at[idx_v.at[j]], rows[b], gsem[b])

        def wait_gather(j, b):
            pltpu.make_async_copy(table_hbm.at[idx_v.at[j]], rows[b], gsem[b]).wait()

        def out_slice(j):
            return out_hbm.at[j, pl.ds(base, bpw)]

        def start_out(j, b):
            pltpu.async_copy(rows[b], out_slice(j), osem[b])

        def wait_out(j, b):
            pltpu.make_async_copy(rows[b], out_slice(j), osem[b]).wait()

        for b in range(NBUF):
            start_gather(b, b)

        def body(jj, carry):
            j0 = jj * NBUF
            for b in range(NBUF):
                wait_gather(j0 + b, b)
                start_out(j0 + b, b)
            for b in range(NBUF):
                wait_out(j0 + b, b)
                start_gather(j0 + NBUF + b, b)
            return carry

        lax.fori_loop(0, seq // NBUF - 1, body, 0)

        jlast = seq - NBUF
        for b in range(NBUF):
            wait_gather(jlast + b, b)
            start_out(jlast + b, b)
        for b in range(NBUF):
            wait_out(jlast + b, b)

    return emb_kernel


def kernel(frame_indices, time_emb_weight):
    batch, seq = frame_indices.shape
    nc, ns = _num_workers()
    nw = nc * ns
    bpw = batch // nw
    idx = frame_indices.astype(jnp.int32).T.reshape(seq, nw, bpw)
    idx = idx.transpose(1, 0, 2)
    out = _build(batch, seq)(time_emb_weight, idx)
    return out.transpose(1, 0, 2)


# speedup vs baseline: 4.0086x; 1.1081x over previous
import functools

import jax
import jax.numpy as jnp
from jax import lax
from jax.experimental import pallas as pl
from jax.experimental.pallas import tpu as pltpu
from jax.experimental.pallas import tpu_sc as plsc

MAX_FRAMES = 256
DIM = 1536
LANES = 16
NBUF = 2
REP = 8


@functools.cache
def _num_workers():
    try:
        info = plsc.get_sparse_core_info()
        return int(info.num_cores), int(info.num_subcores)
    except Exception:
        return 2, 16


@functools.cache
def _build(batch, seq):
    nc, ns = _num_workers()
    nw = nc * ns
    bpw = batch // nw
    mesh = plsc.VectorSubcoreMesh(core_axis_name="c", subcore_axis_name="s")

    @functools.partial(
        pl.kernel,
        mesh=mesh,
        out_type=jax.ShapeDtypeStruct((seq, batch, DIM), jnp.float32),
        scratch_types=[
            pltpu.VMEM((seq, bpw), jnp.int32),
            [pltpu.VMEM((bpw, DIM), jnp.float32) for _ in range(NBUF)],
            [pltpu.SemaphoreType.DMA for _ in range(NBUF)],
            [pltpu.SemaphoreType.DMA for _ in range(NBUF)],
        ],
    )
    def emb_kernel(table_hbm, idx_hbm, out_hbm, idx_v, rows, gsem, osem):
        wid = lax.axis_index("s") * nc + lax.axis_index("c")
        pltpu.sync_copy(idx_hbm.at[wid], idx_v)
        rep_off = (wid % REP) * MAX_FRAMES

        def clamp_row(j, carry):
            for k in range(bpw // LANES):
                v = idx_v[j, pl.ds(k * LANES, LANES)]
                v = jnp.minimum(jnp.maximum(v, 0), MAX_FRAMES - 1)
                idx_v[j, pl.ds(k * LANES, LANES)] = v + rep_off
            return carry

        lax.fori_loop(0, seq, clamp_row, 0)

        base = wid * bpw

        def start_gather(j, b):
            pltpu.async_copy(table_hbm.at[idx_v.at[j]], rows[b], gsem[b])

        def wait_gather(j, b):
            pltpu.make_async_copy(table_hbm.at[idx_v.at[j]], rows[b], gsem[b]).wait()

        def out_slice(j):
            return out_hbm.at[j, pl.ds(base, bpw)]

        def start_out(j, b):
            pltpu.async_copy(rows[b], out_slice(j), osem[b])

        def wait_out(j, b):
            pltpu.make_async_copy(rows[b], out_slice(j), osem[b]).wait()

        for b in range(NBUF):
            start_gather(b, b)

        def body(jj, carry):
            j0 = jj * NBUF
            for b in range(NBUF):
                wait_gather(j0 + b, b)
                start_out(j0 + b, b)
            for b in range(NBUF):
                wait_out(j0 + b, b)
                start_gather(j0 + NBUF + b, b)
            return carry

        lax.fori_loop(0, seq // NBUF - 1, body, 0)

        jlast = seq - NBUF
        for b in range(NBUF):
            wait_gather(jlast + b, b)
            start_out(jlast + b, b)
        for b in range(NBUF):
            wait_out(jlast + b, b)

    return emb_kernel


def kernel(frame_indices, time_emb_weight):
    batch, seq = frame_indices.shape
    nc, ns = _num_workers()
    nw = nc * ns
    bpw = batch // nw
    idx = frame_indices.astype(jnp.int32).T.reshape(seq, nw, bpw)
    idx = idx.transpose(1, 0, 2)
    table = jnp.tile(time_emb_weight, (REP, 1))
    out = _build(batch, seq)(table, idx)
    return out.transpose(1, 0, 2)
